# Initial kernel scaffold; baseline (speedup 1.0000x reference)
import jax
import jax.numpy as jnp
from jax.experimental import pallas as pl


def kernel(x, edge_index, W1, b1, W2, b2, We1, be1, We2, be2):
    raise NotImplementedError("write your pallas kernel here")


# speedup vs baseline: 3.7560x; 3.7560x over previous
import functools

import jax
import jax.numpy as jnp
from jax import lax
from jax.experimental import pallas as pl
from jax.experimental.pallas import tpu as pltpu
from jax.experimental.pallas import tpu_sc as plsc

N = 10000
NP = 10016
D = 256
H = 512
NCHUNK = 4
CW = H // NCHUNK
EPAD = 163840
EPT = EPAD // 32
EPS = EPAD // 16

_mesh = plsc.VectorSubcoreMesh(core_axis_name="c", subcore_axis_name="s")

import dataclasses as _dataclasses
_sc_params = pltpu.CompilerParams()
if "needs_layout_passes" in pltpu.CompilerParams.__dataclass_fields__:
    _sc_params = _dataclasses.replace(_sc_params, needs_layout_passes=False)


@functools.partial(
    pl.kernel,
    out_type=jax.ShapeDtypeStruct((32, NP), jnp.float32),
    mesh=_mesh,
    scratch_types=[
        pltpu.VMEM((EPT,), jnp.int32),
        pltpu.VMEM((NP,), jnp.float32),
    ],
    compiler_params=_sc_params,
)
def _deg_kernel(dst_hbm, out_hbm, idx_v, hist_v):
    c = lax.axis_index("c")
    s = lax.axis_index("s")
    w = s * 2 + c

    @pl.loop(0, NP, step=16)
    def _(i):
        hist_v[pl.ds(i, 16)] = jnp.zeros((16,), jnp.float32)

    pltpu.sync_copy(dst_hbm.at[pl.ds(w * EPT, EPT)], idx_v)
    ones = jnp.ones((16,), jnp.float32)

    @pl.loop(0, EPT, step=16)
    def _(i):
        idx = idx_v[pl.ds(i, 16)]
        plsc.addupdate_scatter(hist_v, [idx], ones)

    pltpu.sync_copy(hist_v, out_hbm.at[w])


def _agg_chunk(m_ref, out_ref, s, acc, srcv, dstv, rows, sem):
    @pl.when(s < 15)
    def _():
        off = s * 632
        pltpu.sync_copy(m_ref.at[pl.ds(off, 632)], acc.at[pl.ds(off, 632)])

    @pl.when(s == 15)
    def _():
        pltpu.sync_copy(m_ref.at[pl.ds(9480, 520)], acc.at[pl.ds(9480, 520)])

    plsc.subcore_barrier()

    @pl.loop(0, EPS // 128)
    def _(b):
        pltpu.async_copy(m_ref.at[srcv.at[b]], rows, sem).wait()
        pltpu.sync_copy(rows, acc.at[dstv.at[b]], add=True)

    plsc.subcore_barrier()

    @pl.when(s < 15)
    def _():
        off = s * 632
        pltpu.sync_copy(acc.at[pl.ds(off, 632)], out_ref.at[pl.ds(off, 632)])

    @pl.when(s == 15)
    def _():
        pltpu.sync_copy(acc.at[pl.ds(9480, 520)], out_ref.at[pl.ds(9480, 520)])

    plsc.subcore_barrier()


@functools.partial(
    pl.kernel,
    out_type=[jax.ShapeDtypeStruct((N, CW), jnp.float32)] * NCHUNK,
    mesh=_mesh,
    scratch_types=[
        pltpu.VMEM_SHARED((NP, CW), jnp.float32),
        pltpu.VMEM((EPS // 128, 128), jnp.int32),
        pltpu.VMEM((EPS // 128, 128), jnp.int32),
        pltpu.VMEM((128, CW), jnp.float32),
        pltpu.SemaphoreType.DMA,
    ],
)
def _agg_kernel(m0, m1, m2, m3, src_hbm, dst_hbm, s0, s1, s2, s3,
                acc, srcv, dstv, rows, sem):
    c = lax.axis_index("c")
    s = lax.axis_index("s")
    pltpu.sync_copy(src_hbm.at[s], srcv)
    pltpu.sync_copy(dst_hbm.at[s], dstv)

    @pl.when(c == 0)
    def _():
        _agg_chunk(m0, s0, s, acc, srcv, dstv, rows, sem)
        _agg_chunk(m1, s1, s, acc, srcv, dstv, rows, sem)

    @pl.when(c == 1)
    def _():
        _agg_chunk(m2, s2, s, acc, srcv, dstv, rows, sem)
        _agg_chunk(m3, s3, s, acc, srcv, dstv, rows, sem)


@functools.partial(
    pl.kernel,
    out_type=jax.ShapeDtypeStruct((EPAD, H), jnp.float32),
    mesh=_mesh,
    scratch_types=[
        pltpu.VMEM((EPT // 64, 64), jnp.int32),
        pltpu.VMEM((EPT // 64, 64), jnp.int32),
        pltpu.VMEM((64, H), jnp.float32),
        pltpu.SemaphoreType.DMA,
    ],
)
def _edge_kernel(hs_hbm, hd_hbm, src_hbm, dst_hbm, z_hbm, srcv, dstv, buf, sem):
    c = lax.axis_index("c")
    s = lax.axis_index("s")
    w = s * 2 + c
    pltpu.sync_copy(src_hbm.at[w], srcv)
    pltpu.sync_copy(dst_hbm.at[w], dstv)

    @pl.loop(0, EPT // 64)
    def _(b):
        pltpu.async_copy(hs_hbm.at[srcv.at[b]], buf, sem).wait()
        pltpu.sync_copy(hd_hbm.at[dstv.at[b]], buf, add=True)
        pltpu.sync_copy(buf, z_hbm.at[pl.ds(w * EPT + b * 64, 64)])


_BM = 1000


def _dinv_of(degt_blk):
    deg = jnp.sum(degt_blk, axis=1, keepdims=True) + 1.0
    return lax.rsqrt(deg)


def _mm1_body(x_ref, w1_ref, degt_ref, o0, o1, o2, o3):
    dinv = _dinv_of(degt_ref[...])
    y = jnp.dot(x_ref[...], w1_ref[...], preferred_element_type=jnp.float32)
    y = y * dinv
    o0[...] = y[:, 0 * CW:1 * CW]
    o1[...] = y[:, 1 * CW:2 * CW]
    o2[...] = y[:, 2 * CW:3 * CW]
    o3[...] = y[:, 3 * CW:4 * CW]


def _mm1(x, w1, degt):
    return pl.pallas_call(
        _mm1_body,
        grid=(N // _BM,),
        in_specs=[
            pl.BlockSpec((_BM, D), lambda i: (i, 0)),
            pl.BlockSpec((D, H), lambda i: (0, 0)),
            pl.BlockSpec((_BM, 32), lambda i: (i, 0)),
        ],
        out_specs=[pl.BlockSpec((_BM, CW), lambda i: (i, 0))] * NCHUNK,
        out_shape=[jax.ShapeDtypeStruct((N, CW), jnp.float32)] * NCHUNK,
    )(x, w1, degt)


def _mm2_body(degt_ref, b_ref, w_ref,
              s0, s1, s2, s3, m0, m1, m2, m3, o0, o1, o2, o3):
    dinv = _dinv_of(degt_ref[...])
    b = b_ref[...]
    acc = jnp.zeros((_BM, H), jnp.float32)
    for kc, (s_ref, m_ref) in enumerate(
            zip((s0, s1, s2, s3), (m0, m1, m2, m3))):
        h = jnp.maximum(
            (s_ref[...] + m_ref[...]) * dinv + b[:, kc * CW:(kc + 1) * CW],
            0.0)
        acc = acc + jnp.dot(h, w_ref[pl.ds(kc * CW, CW), :],
                            preferred_element_type=jnp.float32)
    y = acc * dinv
    o0[...] = y[:, 0 * CW:1 * CW]
    o1[...] = y[:, 1 * CW:2 * CW]
    o2[...] = y[:, 2 * CW:3 * CW]
    o3[...] = y[:, 3 * CW:4 * CW]


def _mm2(degt, b1r, w2, aggs, ms):
    return pl.pallas_call(
        _mm2_body,
        grid=(N // _BM,),
        in_specs=[
            pl.BlockSpec((_BM, 32), lambda i: (i, 0)),
            pl.BlockSpec((1, H), lambda i: (0, 0)),
            pl.BlockSpec((H, H), lambda i: (0, 0)),
        ] + [pl.BlockSpec((_BM, CW), lambda i: (i, 0))] * (2 * NCHUNK),
        out_specs=[pl.BlockSpec((_BM, CW), lambda i: (i, 0))] * NCHUNK,
        out_shape=[jax.ShapeDtypeStruct((N, CW), jnp.float32)] * NCHUNK,
    )(degt, b1r, w2, *aggs, *ms)


def _mm3_body(degt_ref, b_ref, be1_ref, wa_ref, wb_ref,
              s0, s1, s2, s3, m0, m1, m2, m3, hs_ref, hd_ref):
    dinv = _dinv_of(degt_ref[...])
    b = b_ref[...]
    acca = jnp.zeros((_BM, H), jnp.float32)
    accb = jnp.zeros((_BM, H), jnp.float32)
    for kc, (s_ref, m_ref) in enumerate(
            zip((s0, s1, s2, s3), (m0, m1, m2, m3))):
        h = jnp.maximum(
            (s_ref[...] + m_ref[...]) * dinv + b[:, kc * CW:(kc + 1) * CW],
            0.0)
        acca = acca + jnp.dot(h, wa_ref[pl.ds(kc * CW, CW), :],
                              preferred_element_type=jnp.float32)
        accb = accb + jnp.dot(h, wb_ref[pl.ds(kc * CW, CW), :],
                              preferred_element_type=jnp.float32)
    hs_ref[...] = acca
    hd_ref[...] = accb + be1_ref[...]


def _mm3(degt, b2r, be1r, we1a, we1b, aggs, ms):
    return pl.pallas_call(
        _mm3_body,
        grid=(N // _BM,),
        in_specs=[
            pl.BlockSpec((_BM, 32), lambda i: (i, 0)),
            pl.BlockSpec((1, H), lambda i: (0, 0)),
            pl.BlockSpec((1, H), lambda i: (0, 0)),
            pl.BlockSpec((H, H), lambda i: (0, 0)),
            pl.BlockSpec((H, H), lambda i: (0, 0)),
        ] + [pl.BlockSpec((_BM, CW), lambda i: (i, 0))] * (2 * NCHUNK),
        out_specs=[pl.BlockSpec((_BM, H), lambda i: (i, 0))] * 2,
        out_shape=[jax.ShapeDtypeStruct((N, H), jnp.float32)] * 2,
    )(degt, b2r, be1r, we1a, we1b, *aggs, *ms)


_BE = 4096


def _score_body(z_ref, we2_ref, be2_ref, o_ref):
    zr = jnp.maximum(z_ref[...], 0.0)
    logit = jnp.dot(zr, we2_ref[...], preferred_element_type=jnp.float32)
    o_ref[...] = jax.nn.sigmoid(logit + be2_ref[...])


def _score(z, we2, be2r):
    return pl.pallas_call(
        _score_body,
        grid=(EPAD // _BE,),
        in_specs=[
            pl.BlockSpec((_BE, H), lambda i: (i, 0)),
            pl.BlockSpec((H, 1), lambda i: (0, 0)),
            pl.BlockSpec((1, 1), lambda i: (0, 0)),
        ],
        out_specs=pl.BlockSpec((_BE, 1), lambda i: (i, 0)),
        out_shape=jax.ShapeDtypeStruct((EPAD, 1), jnp.float32),
    )(z, we2, be2r)


@jax.jit
def kernel(x, edge_index, W1, b1, W2, b2, We1, be1, We2, be2):
    E = edge_index.shape[1]
    src = edge_index[0]
    dst = edge_index[1]

    pad = EPAD - E
    srcp = jnp.concatenate([src, jnp.zeros((pad,), jnp.int32)])
    dst_sac = jnp.concatenate([dst, jnp.full((pad,), N, jnp.int32)])
    dst_z = jnp.concatenate([dst, jnp.zeros((pad,), jnp.int32)])
    src16 = srcp.reshape(16, EPS // 128, 128)
    dst16 = dst_sac.reshape(16, EPS // 128, 128)
    src32 = srcp.reshape(32, EPT // 64, 64)
    dst32 = dst_z.reshape(32, EPT // 64, 64)

    hist = _deg_kernel(dst_sac)
    degt = jnp.transpose(hist)

    b1r = b1.reshape(1, H)
    b2r = b2.reshape(1, H)
    be1r = be1.reshape(1, H)
    be2r = be2.reshape(1, 1)
    we1a = We1[:H, :]
    we1b = We1[H:, :]

    m1 = _mm1(x, W1, degt)
    agg1 = _agg_kernel(*m1, src16, dst16)
    m2 = _mm2(degt, b1r, W2, agg1, m1)
    agg2 = _agg_kernel(*m2, src16, dst16)
    hs, hd = _mm3(degt, b2r, be1r, we1a, we1b, agg2, m2)

    z = _edge_kernel(hs, hd, src32, dst32)
    scores = _score(z, We2, be2r)
    return scores[:E, 0]


# speedup vs baseline: 4.0179x; 1.0697x over previous
import functools

import jax
import jax.numpy as jnp
from jax import lax
from jax.experimental import pallas as pl
from jax.experimental.pallas import tpu as pltpu
from jax.experimental.pallas import tpu_sc as plsc

N = 10000
NP = 10016
D = 256
H = 512
NCHUNK = 4
CW = H // NCHUNK
EPAD = 163840
EPT = EPAD // 32
EPS = EPAD // 16
AB = 128

_mesh = plsc.VectorSubcoreMesh(core_axis_name="c", subcore_axis_name="s")

import dataclasses as _dataclasses
_sc_params = pltpu.CompilerParams()
if "needs_layout_passes" in pltpu.CompilerParams.__dataclass_fields__:
    _sc_params = _dataclasses.replace(_sc_params, needs_layout_passes=False)


@functools.partial(
    pl.kernel,
    out_type=jax.ShapeDtypeStruct((32, NP), jnp.float32),
    mesh=_mesh,
    scratch_types=[
        pltpu.VMEM((EPT,), jnp.int32),
        pltpu.VMEM((NP,), jnp.float32),
    ],
    compiler_params=_sc_params,
)
def _deg_kernel(dst_hbm, out_hbm, idx_v, hist_v):
    c = lax.axis_index("c")
    s = lax.axis_index("s")
    w = s * 2 + c

    @pl.loop(0, NP, step=16)
    def _(i):
        hist_v[pl.ds(i, 16)] = jnp.zeros((16,), jnp.float32)

    pltpu.sync_copy(dst_hbm.at[pl.ds(w * EPT, EPT)], idx_v)
    ones = jnp.ones((16,), jnp.float32)

    @pl.loop(0, EPT, step=16)
    def _(i):
        idx = idx_v[pl.ds(i, 16)]
        plsc.addupdate_scatter(hist_v, [idx], ones)

    pltpu.sync_copy(hist_v, out_hbm.at[w])


def _agg_chunk(m_ref, out_ref, s, acc, src_hbm, dst_hbm, srcv, dstv,
               rowsA, rowsB, sGA, sGB, sSA, sSB):
    @pl.when(s < 15)
    def _():
        off = s * 632
        pltpu.sync_copy(m_ref.at[pl.ds(off, 632)], acc.at[pl.ds(off, 632)])

    @pl.when(s == 15)
    def _():
        pltpu.sync_copy(m_ref.at[pl.ds(9480, 520)], acc.at[pl.ds(9480, 520)])

    plsc.subcore_barrier()

    NBH = EPS // AB // 2
    for half in range(2):
        pltpu.sync_copy(src_hbm.at[s, pl.ds(half * NBH, NBH)], srcv)
        pltpu.sync_copy(dst_hbm.at[s, pl.ds(half * NBH, NBH)], dstv)
        pltpu.async_copy(m_ref.at[srcv.at[0]], rowsA, sGA)

        @pl.loop(0, NBH, step=2)
        def _(b):
            pltpu.make_async_copy(m_ref.at[srcv.at[b]], rowsA, sGA).wait()

            @pl.when(b > 0)
            def _():
                pltpu.make_async_copy(rowsB, acc.at[dstv.at[b]], sSB).wait()

            pltpu.async_copy(m_ref.at[srcv.at[b + 1]], rowsB, sGB)
            pltpu.async_copy(rowsA, acc.at[dstv.at[b]], sSA, add=True)

            pltpu.make_async_copy(m_ref.at[srcv.at[b + 1]], rowsB, sGB).wait()

            @pl.when(b + 2 < NBH)
            def _():
                pltpu.make_async_copy(rowsA, acc.at[dstv.at[b]], sSA).wait()
                pltpu.async_copy(m_ref.at[srcv.at[b + 2]], rowsA, sGA)

            pltpu.async_copy(rowsB, acc.at[dstv.at[b + 1]], sSB, add=True)

        pltpu.make_async_copy(rowsA, acc.at[dstv.at[0]], sSA).wait()
        pltpu.make_async_copy(rowsB, acc.at[dstv.at[0]], sSB).wait()

    plsc.subcore_barrier()

    @pl.when(s < 15)
    def _():
        off = s * 632
        pltpu.sync_copy(acc.at[pl.ds(off, 632)], out_ref.at[pl.ds(off, 632)])

    @pl.when(s == 15)
    def _():
        pltpu.sync_copy(acc.at[pl.ds(9480, 520)], out_ref.at[pl.ds(9480, 520)])

    plsc.subcore_barrier()


@functools.partial(
    pl.kernel,
    out_type=[jax.ShapeDtypeStruct((N, CW), jnp.float32)] * NCHUNK,
    mesh=_mesh,
    scratch_types=[
        pltpu.VMEM_SHARED((NP, CW), jnp.float32),
        pltpu.VMEM((EPS // AB // 2, AB), jnp.int32),
        pltpu.VMEM((EPS // AB // 2, AB), jnp.int32),
        pltpu.VMEM((AB, CW), jnp.float32),
        pltpu.VMEM((AB, CW), jnp.float32),
        pltpu.SemaphoreType.DMA,
        pltpu.SemaphoreType.DMA,
        pltpu.SemaphoreType.DMA,
        pltpu.SemaphoreType.DMA,
    ],
)
def _agg_kernel(*refs):
    ms = refs[0:NCHUNK]
    src_hbm, dst_hbm = refs[NCHUNK], refs[NCHUNK + 1]
    outs = refs[NCHUNK + 2:2 * NCHUNK + 2]
    acc, srcv, dstv, rowsA, rowsB, sGA, sGB, sSA, sSB = refs[2 * NCHUNK + 2:]
    c = lax.axis_index("c")
    s = lax.axis_index("s")
    half = NCHUNK // 2

    @pl.when(c == 0)
    def _():
        for k in range(half):
            _agg_chunk(ms[k], outs[k], s, acc, src_hbm, dst_hbm, srcv, dstv,
                       rowsA, rowsB, sGA, sGB, sSA, sSB)

    @pl.when(c == 1)
    def _():
        for k in range(half, NCHUNK):
            _agg_chunk(ms[k], outs[k], s, acc, src_hbm, dst_hbm, srcv, dstv,
                       rowsA, rowsB, sGA, sGB, sSA, sSB)


@functools.partial(
    pl.kernel,
    out_type=jax.ShapeDtypeStruct((EPAD, H), jnp.float32),
    mesh=_mesh,
    scratch_types=[
        pltpu.VMEM((EPT // 64, 64), jnp.int32),
        pltpu.VMEM((EPT // 64, 64), jnp.int32),
        pltpu.VMEM((64, H), jnp.float32),
        pltpu.SemaphoreType.DMA,
    ],
)
def _edge_kernel(hs_hbm, hd_hbm, src_hbm, dst_hbm, z_hbm, srcv, dstv, buf, sem):
    c = lax.axis_index("c")
    s = lax.axis_index("s")
    w = s * 2 + c
    pltpu.sync_copy(src_hbm.at[w], srcv)
    pltpu.sync_copy(dst_hbm.at[w], dstv)

    @pl.loop(0, EPT // 64)
    def _(b):
        pltpu.async_copy(hs_hbm.at[srcv.at[b]], buf, sem).wait()
        pltpu.sync_copy(hd_hbm.at[dstv.at[b]], buf, add=True)
        pltpu.sync_copy(buf, z_hbm.at[pl.ds(w * EPT + b * 64, 64)])


_BM = 1000


def _dinv_of(degt_blk):
    deg = jnp.sum(degt_blk, axis=1, keepdims=True) + 1.0
    return lax.rsqrt(deg)


def _mm1_body(x_ref, w1_ref, degt_ref, *outs):
    dinv = _dinv_of(degt_ref[...])
    y = jnp.dot(x_ref[...], w1_ref[...], preferred_element_type=jnp.float32)
    y = y * dinv
    for kc, o in enumerate(outs):
        o[...] = y[:, kc * CW:(kc + 1) * CW]


def _mm1(x, w1, degt):
    return pl.pallas_call(
        _mm1_body,
        grid=(N // _BM,),
        in_specs=[
            pl.BlockSpec((_BM, D), lambda i: (i, 0)),
            pl.BlockSpec((D, H), lambda i: (0, 0)),
            pl.BlockSpec((_BM, 32), lambda i: (i, 0)),
        ],
        out_specs=[pl.BlockSpec((_BM, CW), lambda i: (i, 0))] * NCHUNK,
        out_shape=[jax.ShapeDtypeStruct((N, CW), jnp.float32)] * NCHUNK,
    )(x, w1, degt)


def _mm2_body(degt_ref, b_ref, w_ref, *rest):
    ss = rest[0:NCHUNK]
    mm = rest[NCHUNK:2 * NCHUNK]
    outs = rest[2 * NCHUNK:]
    dinv = _dinv_of(degt_ref[...])
    b = b_ref[...]
    acc = jnp.zeros((_BM, H), jnp.float32)
    for kc in range(NCHUNK):
        h = jnp.maximum(
            (ss[kc][...] + mm[kc][...]) * dinv + b[:, kc * CW:(kc + 1) * CW],
            0.0)
        acc = acc + jnp.dot(h, w_ref[pl.ds(kc * CW, CW), :],
                            preferred_element_type=jnp.float32)
    y = acc * dinv
    for kc, o in enumerate(outs):
        o[...] = y[:, kc * CW:(kc + 1) * CW]


def _mm2(degt, b1r, w2, aggs, ms):
    return pl.pallas_call(
        _mm2_body,
        grid=(N // _BM,),
        in_specs=[
            pl.BlockSpec((_BM, 32), lambda i: (i, 0)),
            pl.BlockSpec((1, H), lambda i: (0, 0)),
            pl.BlockSpec((H, H), lambda i: (0, 0)),
        ] + [pl.BlockSpec((_BM, CW), lambda i: (i, 0))] * (2 * NCHUNK),
        out_specs=[pl.BlockSpec((_BM, CW), lambda i: (i, 0))] * NCHUNK,
        out_shape=[jax.ShapeDtypeStruct((N, CW), jnp.float32)] * NCHUNK,
    )(degt, b1r, w2, *aggs, *ms)


def _mm3_body(degt_ref, b_ref, be1_ref, wa_ref, wb_ref, *rest):
    ss = rest[0:NCHUNK]
    mm = rest[NCHUNK:2 * NCHUNK]
    hs_ref, hd_ref = rest[2 * NCHUNK:]
    dinv = _dinv_of(degt_ref[...])
    b = b_ref[...]
    acca = jnp.zeros((_BM, H), jnp.float32)
    accb = jnp.zeros((_BM, H), jnp.float32)
    for kc in range(NCHUNK):
        h = jnp.maximum(
            (ss[kc][...] + mm[kc][...]) * dinv + b[:, kc * CW:(kc + 1) * CW],
            0.0)
        acca = acca + jnp.dot(h, wa_ref[pl.ds(kc * CW, CW), :],
                              preferred_element_type=jnp.float32)
        accb = accb + jnp.dot(h, wb_ref[pl.ds(kc * CW, CW), :],
                              preferred_element_type=jnp.float32)
    hs_ref[...] = acca
    hd_ref[...] = accb + be1_ref[...]


def _mm3(degt, b2r, be1r, we1a, we1b, aggs, ms):
    return pl.pallas_call(
        _mm3_body,
        grid=(N // _BM,),
        in_specs=[
            pl.BlockSpec((_BM, 32), lambda i: (i, 0)),
            pl.BlockSpec((1, H), lambda i: (0, 0)),
            pl.BlockSpec((1, H), lambda i: (0, 0)),
            pl.BlockSpec((H, H), lambda i: (0, 0)),
            pl.BlockSpec((H, H), lambda i: (0, 0)),
        ] + [pl.BlockSpec((_BM, CW), lambda i: (i, 0))] * (2 * NCHUNK),
        out_specs=[pl.BlockSpec((_BM, H), lambda i: (i, 0))] * 2,
        out_shape=[jax.ShapeDtypeStruct((N, H), jnp.float32)] * 2,
    )(degt, b2r, be1r, we1a, we1b, *aggs, *ms)


_BE = 4096


def _score_body(z_ref, we2_ref, be2_ref, o_ref):
    zr = jnp.maximum(z_ref[...], 0.0)
    logit = jnp.dot(zr, we2_ref[...], preferred_element_type=jnp.float32)
    o_ref[...] = jax.nn.sigmoid(logit + be2_ref[...])


def _score(z, we2, be2r):
    return pl.pallas_call(
        _score_body,
        grid=(EPAD // _BE,),
        in_specs=[
            pl.BlockSpec((_BE, H), lambda i: (i, 0)),
            pl.BlockSpec((H, 1), lambda i: (0, 0)),
            pl.BlockSpec((1, 1), lambda i: (0, 0)),
        ],
        out_specs=pl.BlockSpec((_BE, 1), lambda i: (i, 0)),
        out_shape=jax.ShapeDtypeStruct((EPAD, 1), jnp.float32),
    )(z, we2, be2r)


@jax.jit
def kernel(x, edge_index, W1, b1, W2, b2, We1, be1, We2, be2):
    E = edge_index.shape[1]
    src = edge_index[0]
    dst = edge_index[1]

    pad = EPAD - E
    srcp = jnp.concatenate([src, jnp.zeros((pad,), jnp.int32)])
    dst_sac = jnp.concatenate([dst, jnp.full((pad,), N, jnp.int32)])
    dst_z = jnp.concatenate([dst, jnp.zeros((pad,), jnp.int32)])
    src16 = srcp.reshape(16, EPS // AB, AB)
    dst16 = dst_sac.reshape(16, EPS // AB, AB)
    src32 = srcp.reshape(32, EPT // 64, 64)
    dst32 = dst_z.reshape(32, EPT // 64, 64)

    hist = _deg_kernel(dst_sac)
    degt = jnp.transpose(hist)

    b1r = b1.reshape(1, H)
    b2r = b2.reshape(1, H)
    be1r = be1.reshape(1, H)
    be2r = be2.reshape(1, 1)
    we1a = We1[:H, :]
    we1b = We1[H:, :]

    m1 = _mm1(x, W1, degt)
    agg1 = _agg_kernel(*m1, src16, dst16)
    m2 = _mm2(degt, b1r, W2, agg1, m1)
    agg2 = _agg_kernel(*m2, src16, dst16)
    hs, hd = _mm3(degt, b2r, be1r, we1a, we1b, agg2, m2)

    z = _edge_kernel(hs, hd, src32, dst32)
    scores = _score(z, We2, be2r)
    return scores[:E, 0]


# speedup vs baseline: 4.5746x; 1.1385x over previous
import functools

import jax
import jax.numpy as jnp
from jax import lax
from jax.experimental import pallas as pl
from jax.experimental.pallas import tpu as pltpu
from jax.experimental.pallas import tpu_sc as plsc

N = 10000
NP = 10016
D = 256
H = 512
NCHUNK = 4
CW = H // NCHUNK
EPAD = 163840
EPT = EPAD // 32
EPS = EPAD // 16
AB = 128

_mesh = plsc.VectorSubcoreMesh(core_axis_name="c", subcore_axis_name="s")

import dataclasses as _dataclasses
_sc_params = pltpu.CompilerParams()
if "needs_layout_passes" in pltpu.CompilerParams.__dataclass_fields__:
    _sc_params = _dataclasses.replace(_sc_params, needs_layout_passes=False)


@functools.partial(
    pl.kernel,
    out_type=jax.ShapeDtypeStruct((32, NP), jnp.float32),
    mesh=_mesh,
    scratch_types=[
        pltpu.VMEM((EPT,), jnp.int32),
        pltpu.VMEM((NP,), jnp.float32),
    ],
    compiler_params=_sc_params,
)
def _deg_kernel(dst_hbm, out_hbm, idx_v, hist_v):
    c = lax.axis_index("c")
    s = lax.axis_index("s")
    w = s * 2 + c

    @pl.loop(0, NP, step=16)
    def _(i):
        hist_v[pl.ds(i, 16)] = jnp.zeros((16,), jnp.float32)

    pltpu.sync_copy(dst_hbm.at[pl.ds(w * EPT, EPT)], idx_v)
    ones = jnp.ones((16,), jnp.float32)

    @pl.loop(0, EPT, step=16)
    def _(i):
        idx = idx_v[pl.ds(i, 16)]
        plsc.addupdate_scatter(hist_v, [idx], ones)

    pltpu.sync_copy(hist_v, out_hbm.at[w])


def _agg_chunk(m_ref, out_ref, s, acc, src_hbm, dst_hbm, srcv, dstv,
               rowsA, rowsB, sGA, sGB, sSA, sSB):
    @pl.when(s < 15)
    def _():
        off = s * 632
        pltpu.sync_copy(m_ref.at[pl.ds(off, 632)], acc.at[pl.ds(off, 632)])

    @pl.when(s == 15)
    def _():
        pltpu.sync_copy(m_ref.at[pl.ds(9480, 520)], acc.at[pl.ds(9480, 520)])

    plsc.subcore_barrier()

    NBH = EPS // AB // 2
    for half in range(2):
        pltpu.sync_copy(src_hbm.at[s, pl.ds(half * NBH, NBH)], srcv)
        pltpu.sync_copy(dst_hbm.at[s, pl.ds(half * NBH, NBH)], dstv)
        pltpu.async_copy(m_ref.at[srcv.at[0]], rowsA, sGA)

        @pl.loop(0, NBH, step=2)
        def _(b):
            pltpu.make_async_copy(m_ref.at[srcv.at[b]], rowsA, sGA).wait()

            @pl.when(b > 0)
            def _():
                pltpu.make_async_copy(rowsB, acc.at[dstv.at[b]], sSB).wait()

            pltpu.async_copy(m_ref.at[srcv.at[b + 1]], rowsB, sGB)
            pltpu.async_copy(rowsA, acc.at[dstv.at[b]], sSA, add=True)

            pltpu.make_async_copy(m_ref.at[srcv.at[b + 1]], rowsB, sGB).wait()

            @pl.when(b + 2 < NBH)
            def _():
                pltpu.make_async_copy(rowsA, acc.at[dstv.at[b]], sSA).wait()
                pltpu.async_copy(m_ref.at[srcv.at[b + 2]], rowsA, sGA)

            pltpu.async_copy(rowsB, acc.at[dstv.at[b + 1]], sSB, add=True)

        pltpu.make_async_copy(rowsA, acc.at[dstv.at[0]], sSA).wait()
        pltpu.make_async_copy(rowsB, acc.at[dstv.at[0]], sSB).wait()

    plsc.subcore_barrier()

    @pl.when(s < 15)
    def _():
        off = s * 632
        pltpu.sync_copy(acc.at[pl.ds(off, 632)], out_ref.at[pl.ds(off, 632)])

    @pl.when(s == 15)
    def _():
        pltpu.sync_copy(acc.at[pl.ds(9480, 520)], out_ref.at[pl.ds(9480, 520)])

    plsc.subcore_barrier()


@functools.partial(
    pl.kernel,
    out_type=[jax.ShapeDtypeStruct((N, CW), jnp.float32)] * NCHUNK,
    mesh=_mesh,
    scratch_types=[
        pltpu.VMEM_SHARED((NP, CW), jnp.float32),
        pltpu.VMEM((EPS // AB // 2, AB), jnp.int32),
        pltpu.VMEM((EPS // AB // 2, AB), jnp.int32),
        pltpu.VMEM((AB, CW), jnp.float32),
        pltpu.VMEM((AB, CW), jnp.float32),
        pltpu.SemaphoreType.DMA,
        pltpu.SemaphoreType.DMA,
        pltpu.SemaphoreType.DMA,
        pltpu.SemaphoreType.DMA,
    ],
)
def _agg_kernel(*refs):
    ms = refs[0:NCHUNK]
    src_hbm, dst_hbm = refs[NCHUNK], refs[NCHUNK + 1]
    outs = refs[NCHUNK + 2:2 * NCHUNK + 2]
    acc, srcv, dstv, rowsA, rowsB, sGA, sGB, sSA, sSB = refs[2 * NCHUNK + 2:]
    c = lax.axis_index("c")
    s = lax.axis_index("s")
    half = NCHUNK // 2

    @pl.when(c == 0)
    def _():
        for k in range(half):
            _agg_chunk(ms[k], outs[k], s, acc, src_hbm, dst_hbm, srcv, dstv,
                       rowsA, rowsB, sGA, sGB, sSA, sSB)

    @pl.when(c == 1)
    def _():
        for k in range(half, NCHUNK):
            _agg_chunk(ms[k], outs[k], s, acc, src_hbm, dst_hbm, srcv, dstv,
                       rowsA, rowsB, sGA, sGB, sSA, sSB)


@functools.partial(
    pl.kernel,
    out_type=jax.ShapeDtypeStruct((EPAD, H), jnp.float32),
    mesh=_mesh,
    scratch_types=[
        pltpu.VMEM((EPT // 64, 64), jnp.int32),
        pltpu.VMEM((EPT // 64, 64), jnp.int32),
        pltpu.VMEM((64, H), jnp.float32),
        pltpu.VMEM((64, H), jnp.float32),
        pltpu.SemaphoreType.DMA,
        pltpu.SemaphoreType.DMA,
        pltpu.SemaphoreType.DMA,
        pltpu.SemaphoreType.DMA,
        pltpu.SemaphoreType.DMA,
        pltpu.SemaphoreType.DMA,
    ],
)
def _edge_kernel(hs_hbm, hd_hbm, src_hbm, dst_hbm, z_hbm, srcv, dstv,
                 bufA, bufB, sGA, sGB, sHA, sHB, sWA, sWB):
    c = lax.axis_index("c")
    s = lax.axis_index("s")
    w = s * 2 + c
    pltpu.sync_copy(src_hbm.at[w], srcv)
    pltpu.sync_copy(dst_hbm.at[w], dstv)

    NB = EPT // 64
    pltpu.async_copy(hs_hbm.at[srcv.at[0]], bufA, sGA)

    @pl.loop(0, NB, step=2)
    def _(b):
        pltpu.make_async_copy(hs_hbm.at[srcv.at[b]], bufA, sGA).wait()
        pltpu.async_copy(hd_hbm.at[dstv.at[b]], bufA, sHA, add=True)

        @pl.when(b > 0)
        def _():
            pltpu.make_async_copy(bufB, z_hbm.at[pl.ds(w * EPT, 64)],
                                  sWB).wait()

        pltpu.async_copy(hs_hbm.at[srcv.at[b + 1]], bufB, sGB)
        pltpu.make_async_copy(hd_hbm.at[dstv.at[b]], bufA, sHA).wait()
        pltpu.async_copy(bufA, z_hbm.at[pl.ds(w * EPT + b * 64, 64)], sWA)

        pltpu.make_async_copy(hs_hbm.at[srcv.at[b + 1]], bufB, sGB).wait()
        pltpu.async_copy(hd_hbm.at[dstv.at[b + 1]], bufB, sHB, add=True)

        @pl.when(b + 2 < NB)
        def _():
            pltpu.make_async_copy(bufA, z_hbm.at[pl.ds(w * EPT, 64)],
                                  sWA).wait()
            pltpu.async_copy(hs_hbm.at[srcv.at[b + 2]], bufA, sGA)

        pltpu.make_async_copy(hd_hbm.at[dstv.at[b + 1]], bufB, sHB).wait()
        pltpu.async_copy(bufB, z_hbm.at[pl.ds(w * EPT + (b + 1) * 64, 64)],
                         sWB)

    pltpu.make_async_copy(bufA, z_hbm.at[pl.ds(w * EPT, 64)], sWA).wait()
    pltpu.make_async_copy(bufB, z_hbm.at[pl.ds(w * EPT, 64)], sWB).wait()


_BM = 1000


def _dinv_of(degt_blk):
    deg = jnp.sum(degt_blk, axis=1, keepdims=True) + 1.0
    return lax.rsqrt(deg)


def _mm1_body(x_ref, w1_ref, degt_ref, *outs):
    dinv = _dinv_of(degt_ref[...])
    y = jnp.dot(x_ref[...], w1_ref[...], preferred_element_type=jnp.float32)
    y = y * dinv
    for kc, o in enumerate(outs):
        o[...] = y[:, kc * CW:(kc + 1) * CW]


def _mm1(x, w1, degt):
    return pl.pallas_call(
        _mm1_body,
        grid=(N // _BM,),
        in_specs=[
            pl.BlockSpec((_BM, D), lambda i: (i, 0)),
            pl.BlockSpec((D, H), lambda i: (0, 0)),
            pl.BlockSpec((_BM, 32), lambda i: (i, 0)),
        ],
        out_specs=[pl.BlockSpec((_BM, CW), lambda i: (i, 0))] * NCHUNK,
        out_shape=[jax.ShapeDtypeStruct((N, CW), jnp.float32)] * NCHUNK,
    )(x, w1, degt)


def _mm2_body(degt_ref, b_ref, w_ref, *rest):
    ss = rest[0:NCHUNK]
    mm = rest[NCHUNK:2 * NCHUNK]
    outs = rest[2 * NCHUNK:]
    dinv = _dinv_of(degt_ref[...])
    b = b_ref[...]
    acc = jnp.zeros((_BM, H), jnp.float32)
    for kc in range(NCHUNK):
        h = jnp.maximum(
            (ss[kc][...] + mm[kc][...]) * dinv + b[:, kc * CW:(kc + 1) * CW],
            0.0)
        acc = acc + jnp.dot(h, w_ref[pl.ds(kc * CW, CW), :],
                            preferred_element_type=jnp.float32)
    y = acc * dinv
    for kc, o in enumerate(outs):
        o[...] = y[:, kc * CW:(kc + 1) * CW]


def _mm2(degt, b1r, w2, aggs, ms):
    return pl.pallas_call(
        _mm2_body,
        grid=(N // _BM,),
        in_specs=[
            pl.BlockSpec((_BM, 32), lambda i: (i, 0)),
            pl.BlockSpec((1, H), lambda i: (0, 0)),
            pl.BlockSpec((H, H), lambda i: (0, 0)),
        ] + [pl.BlockSpec((_BM, CW), lambda i: (i, 0))] * (2 * NCHUNK),
        out_specs=[pl.BlockSpec((_BM, CW), lambda i: (i, 0))] * NCHUNK,
        out_shape=[jax.ShapeDtypeStruct((N, CW), jnp.float32)] * NCHUNK,
    )(degt, b1r, w2, *aggs, *ms)


def _mm3_body(degt_ref, b_ref, be1_ref, wa_ref, wb_ref, *rest):
    ss = rest[0:NCHUNK]
    mm = rest[NCHUNK:2 * NCHUNK]
    hs_ref, hd_ref = rest[2 * NCHUNK:]
    dinv = _dinv_of(degt_ref[...])
    b = b_ref[...]
    acca = jnp.zeros((_BM, H), jnp.float32)
    accb = jnp.zeros((_BM, H), jnp.float32)
    for kc in range(NCHUNK):
        h = jnp.maximum(
            (ss[kc][...] + mm[kc][...]) * dinv + b[:, kc * CW:(kc + 1) * CW],
            0.0)
        acca = acca + jnp.dot(h, wa_ref[pl.ds(kc * CW, CW), :],
                              preferred_element_type=jnp.float32)
        accb = accb + jnp.dot(h, wb_ref[pl.ds(kc * CW, CW), :],
                              preferred_element_type=jnp.float32)
    hs_ref[...] = acca
    hd_ref[...] = accb + be1_ref[...]


def _mm3(degt, b2r, be1r, we1a, we1b, aggs, ms):
    return pl.pallas_call(
        _mm3_body,
        grid=(N // _BM,),
        in_specs=[
            pl.BlockSpec((_BM, 32), lambda i: (i, 0)),
            pl.BlockSpec((1, H), lambda i: (0, 0)),
            pl.BlockSpec((1, H), lambda i: (0, 0)),
            pl.BlockSpec((H, H), lambda i: (0, 0)),
            pl.BlockSpec((H, H), lambda i: (0, 0)),
        ] + [pl.BlockSpec((_BM, CW), lambda i: (i, 0))] * (2 * NCHUNK),
        out_specs=[pl.BlockSpec((_BM, H), lambda i: (i, 0))] * 2,
        out_shape=[jax.ShapeDtypeStruct((N, H), jnp.float32)] * 2,
    )(degt, b2r, be1r, we1a, we1b, *aggs, *ms)


_BE = 4096


def _score_body(z_ref, we2_ref, be2_ref, o_ref):
    zr = jnp.maximum(z_ref[...], 0.0)
    logit = jnp.dot(zr, we2_ref[...], preferred_element_type=jnp.float32)
    o_ref[...] = jax.nn.sigmoid(logit + be2_ref[...])


def _score(z, we2, be2r):
    return pl.pallas_call(
        _score_body,
        grid=(EPAD // _BE,),
        in_specs=[
            pl.BlockSpec((_BE, H), lambda i: (i, 0)),
            pl.BlockSpec((H, 1), lambda i: (0, 0)),
            pl.BlockSpec((1, 1), lambda i: (0, 0)),
        ],
        out_specs=pl.BlockSpec((_BE, 1), lambda i: (i, 0)),
        out_shape=jax.ShapeDtypeStruct((EPAD, 1), jnp.float32),
    )(z, we2, be2r)


@jax.jit
def kernel(x, edge_index, W1, b1, W2, b2, We1, be1, We2, be2):
    E = edge_index.shape[1]
    src = edge_index[0]
    dst = edge_index[1]

    pad = EPAD - E
    srcp = jnp.concatenate([src, jnp.zeros((pad,), jnp.int32)])
    dst_sac = jnp.concatenate([dst, jnp.full((pad,), N, jnp.int32)])
    dst_z = jnp.concatenate([dst, jnp.zeros((pad,), jnp.int32)])
    src16 = srcp.reshape(16, EPS // AB, AB)
    dst16 = dst_sac.reshape(16, EPS // AB, AB)
    src32 = srcp.reshape(32, EPT // 64, 64)
    dst32 = dst_z.reshape(32, EPT // 64, 64)

    hist = _deg_kernel(dst_sac)
    degt = jnp.transpose(hist)

    b1r = b1.reshape(1, H)
    b2r = b2.reshape(1, H)
    be1r = be1.reshape(1, H)
    be2r = be2.reshape(1, 1)
    we1a = We1[:H, :]
    we1b = We1[H:, :]

    m1 = _mm1(x, W1, degt)
    agg1 = _agg_kernel(*m1, src16, dst16)
    m2 = _mm2(degt, b1r, W2, agg1, m1)
    agg2 = _agg_kernel(*m2, src16, dst16)
    hs, hd = _mm3(degt, b2r, be1r, we1a, we1b, agg2, m2)

    z = _edge_kernel(hs, hd, src32, dst32)
    scores = _score(z, We2, be2r)
    return scores[:E, 0]


# speedup vs baseline: 4.6126x; 1.0083x over previous
import functools

import jax
import jax.numpy as jnp
from jax import lax
from jax.experimental import pallas as pl
from jax.experimental.pallas import tpu as pltpu
from jax.experimental.pallas import tpu_sc as plsc

N = 10000
NP = 10016
D = 256
H = 512
NCHUNK = 4
CW = H // NCHUNK
EPAD = 163840
EPT = EPAD // 32
EPS = EPAD // 16
AB = 128

_mesh = plsc.VectorSubcoreMesh(core_axis_name="c", subcore_axis_name="s")

import dataclasses as _dataclasses
_sc_params = pltpu.CompilerParams()
if "needs_layout_passes" in pltpu.CompilerParams.__dataclass_fields__:
    _sc_params = _dataclasses.replace(_sc_params, needs_layout_passes=False)


@functools.partial(
    pl.kernel,
    out_type=jax.ShapeDtypeStruct((32, NP), jnp.float32),
    mesh=_mesh,
    scratch_types=[
        pltpu.VMEM((EPT,), jnp.int32),
        pltpu.VMEM((NP,), jnp.float32),
    ],
    compiler_params=_sc_params,
)
def _deg_kernel(dst_hbm, out_hbm, idx_v, hist_v):
    c = lax.axis_index("c")
    s = lax.axis_index("s")
    w = s * 2 + c

    @pl.loop(0, NP, step=16)
    def _(i):
        hist_v[pl.ds(i, 16)] = jnp.zeros((16,), jnp.float32)

    pltpu.sync_copy(dst_hbm.at[pl.ds(w * EPT, EPT)], idx_v)
    ones = jnp.ones((16,), jnp.float32)

    @pl.loop(0, EPT, step=16)
    def _(i):
        idx = idx_v[pl.ds(i, 16)]
        plsc.addupdate_scatter(hist_v, [idx], ones)

    pltpu.sync_copy(hist_v, out_hbm.at[w])


def _agg_chunk(m_ref, out_ref, s, acc, src_hbm, dst_hbm, srcv, dstv,
               rowsA, rowsB, sGA, sGB, sSA, sSB):
    @pl.when(s < 15)
    def _():
        off = s * 632
        pltpu.sync_copy(m_ref.at[pl.ds(off, 632)], acc.at[pl.ds(off, 632)])

    @pl.when(s == 15)
    def _():
        pltpu.sync_copy(m_ref.at[pl.ds(9480, 520)], acc.at[pl.ds(9480, 520)])

    plsc.subcore_barrier()

    NBH = EPS // AB // 2
    for half in range(2):
        pltpu.sync_copy(src_hbm.at[s, pl.ds(half * NBH, NBH)], srcv)
        pltpu.sync_copy(dst_hbm.at[s, pl.ds(half * NBH, NBH)], dstv)
        pltpu.async_copy(m_ref.at[srcv.at[0]], rowsA, sGA)

        @pl.loop(0, NBH, step=2)
        def _(b):
            pltpu.make_async_copy(m_ref.at[srcv.at[b]], rowsA, sGA).wait()

            @pl.when(b > 0)
            def _():
                pltpu.make_async_copy(rowsB, acc.at[dstv.at[b]], sSB).wait()

            pltpu.async_copy(m_ref.at[srcv.at[b + 1]], rowsB, sGB)
            pltpu.async_copy(rowsA, acc.at[dstv.at[b]], sSA, add=True)

            pltpu.make_async_copy(m_ref.at[srcv.at[b + 1]], rowsB, sGB).wait()

            @pl.when(b + 2 < NBH)
            def _():
                pltpu.make_async_copy(rowsA, acc.at[dstv.at[b]], sSA).wait()
                pltpu.async_copy(m_ref.at[srcv.at[b + 2]], rowsA, sGA)

            pltpu.async_copy(rowsB, acc.at[dstv.at[b + 1]], sSB, add=True)

        pltpu.make_async_copy(rowsA, acc.at[dstv.at[0]], sSA).wait()
        pltpu.make_async_copy(rowsB, acc.at[dstv.at[0]], sSB).wait()

    plsc.subcore_barrier()

    @pl.when(s < 15)
    def _():
        off = s * 632
        pltpu.sync_copy(acc.at[pl.ds(off, 632)], out_ref.at[pl.ds(off, 632)])

    @pl.when(s == 15)
    def _():
        pltpu.sync_copy(acc.at[pl.ds(9480, 520)], out_ref.at[pl.ds(9480, 520)])

    plsc.subcore_barrier()


@functools.partial(
    pl.kernel,
    out_type=[jax.ShapeDtypeStruct((N, CW), jnp.float32)] * NCHUNK,
    mesh=_mesh,
    scratch_types=[
        pltpu.VMEM_SHARED((NP, CW), jnp.float32),
        pltpu.VMEM((EPS // AB // 2, AB), jnp.int32),
        pltpu.VMEM((EPS // AB // 2, AB), jnp.int32),
        pltpu.VMEM((AB, CW), jnp.float32),
        pltpu.VMEM((AB, CW), jnp.float32),
        pltpu.SemaphoreType.DMA,
        pltpu.SemaphoreType.DMA,
        pltpu.SemaphoreType.DMA,
        pltpu.SemaphoreType.DMA,
    ],
)
def _agg_kernel(*refs):
    ms = refs[0:NCHUNK]
    src_hbm, dst_hbm = refs[NCHUNK], refs[NCHUNK + 1]
    outs = refs[NCHUNK + 2:2 * NCHUNK + 2]
    acc, srcv, dstv, rowsA, rowsB, sGA, sGB, sSA, sSB = refs[2 * NCHUNK + 2:]
    c = lax.axis_index("c")
    s = lax.axis_index("s")
    half = NCHUNK // 2

    @pl.when(c == 0)
    def _():
        for k in range(half):
            _agg_chunk(ms[k], outs[k], s, acc, src_hbm, dst_hbm, srcv, dstv,
                       rowsA, rowsB, sGA, sGB, sSA, sSB)

    @pl.when(c == 1)
    def _():
        for k in range(half, NCHUNK):
            _agg_chunk(ms[k], outs[k], s, acc, src_hbm, dst_hbm, srcv, dstv,
                       rowsA, rowsB, sGA, sGB, sSA, sSB)


@functools.partial(
    pl.kernel,
    out_type=jax.ShapeDtypeStruct((EPAD,), jnp.float32),
    mesh=_mesh,
    scratch_types=[
        pltpu.VMEM((EPT // 64, 64), jnp.int32),
        pltpu.VMEM((EPT // 64, 64), jnp.int32),
        pltpu.VMEM((64, H), jnp.float32),
        pltpu.VMEM((64, H), jnp.float32),
        pltpu.VMEM((H,), jnp.float32),
        pltpu.VMEM((16,), jnp.float32),
        pltpu.VMEM((EPT,), jnp.float32),
        pltpu.SemaphoreType.DMA,
        pltpu.SemaphoreType.DMA,
        pltpu.SemaphoreType.DMA,
        pltpu.SemaphoreType.DMA,
    ],
    compiler_params=_sc_params,
)
def _edge_kernel(hs_hbm, hd_hbm, src_hbm, dst_hbm, we2_hbm, be2_hbm, out_hbm,
                 srcv, dstv, bufA, bufB, wv, b2v, score_v,
                 sGA, sGB, sHA, sHB):
    c = lax.axis_index("c")
    s = lax.axis_index("s")
    w = s * 2 + c
    pltpu.sync_copy(src_hbm.at[w], srcv)
    pltpu.sync_copy(dst_hbm.at[w], dstv)
    pltpu.sync_copy(we2_hbm, wv)
    pltpu.sync_copy(be2_hbm, b2v)

    wregs = [wv[pl.ds(kc * 16, 16)] for kc in range(H // 16)]
    lane0 = lax.iota(jnp.int32, 16) == 0

    def compute(buf, batch):
        base = batch * 64

        @pl.loop(0, 64)
        def _(e):
            acc = jnp.zeros((16,), jnp.float32)
            for kc in range(H // 16):
                u = jnp.maximum(buf[e, pl.ds(kc * 16, 16)], 0.0)
                acc = acc + u * wregs[kc]
            t = jnp.broadcast_to(jnp.sum(acc), (16,))
            idx = jnp.broadcast_to(base + e, (16,)).astype(jnp.int32)
            plsc.store_scatter(score_v, [idx], t, mask=lane0)

    NB = EPT // 64
    pltpu.async_copy(hs_hbm.at[srcv.at[0]], bufA, sGA)

    @pl.loop(0, NB, step=2)
    def _(b):
        pltpu.make_async_copy(hs_hbm.at[srcv.at[b]], bufA, sGA).wait()
        pltpu.async_copy(hd_hbm.at[dstv.at[b]], bufA, sHA, add=True)
        pltpu.async_copy(hs_hbm.at[srcv.at[b + 1]], bufB, sGB)
        pltpu.make_async_copy(hd_hbm.at[dstv.at[b]], bufA, sHA).wait()
        compute(bufA, b)

        pltpu.make_async_copy(hs_hbm.at[srcv.at[b + 1]], bufB, sGB).wait()
        pltpu.async_copy(hd_hbm.at[dstv.at[b + 1]], bufB, sHB, add=True)

        @pl.when(b + 2 < NB)
        def _():
            pltpu.async_copy(hs_hbm.at[srcv.at[b + 2]], bufA, sGA)

        pltpu.make_async_copy(hd_hbm.at[dstv.at[b + 1]], bufB, sHB).wait()
        compute(bufB, b + 1)

    b2 = b2v[pl.ds(0, 16)]

    @pl.loop(0, EPT, step=16)
    def _(i):
        t = score_v[pl.ds(i, 16)] + b2
        score_v[pl.ds(i, 16)] = 1.0 / (1.0 + jnp.exp(-t))

    pltpu.sync_copy(score_v, out_hbm.at[pl.ds(w * EPT, EPT)])


_BM = 1000


def _dinv_of(degt_blk):
    deg = jnp.sum(degt_blk, axis=1, keepdims=True) + 1.0
    return lax.rsqrt(deg)


def _mm1_body(x_ref, w1_ref, degt_ref, *outs):
    dinv = _dinv_of(degt_ref[...])
    y = jnp.dot(x_ref[...], w1_ref[...], preferred_element_type=jnp.float32)
    y = y * dinv
    for kc, o in enumerate(outs):
        o[...] = y[:, kc * CW:(kc + 1) * CW]


def _mm1(x, w1, degt):
    return pl.pallas_call(
        _mm1_body,
        grid=(N // _BM,),
        in_specs=[
            pl.BlockSpec((_BM, D), lambda i: (i, 0)),
            pl.BlockSpec((D, H), lambda i: (0, 0)),
            pl.BlockSpec((_BM, 32), lambda i: (i, 0)),
        ],
        out_specs=[pl.BlockSpec((_BM, CW), lambda i: (i, 0))] * NCHUNK,
        out_shape=[jax.ShapeDtypeStruct((N, CW), jnp.float32)] * NCHUNK,
    )(x, w1, degt)


def _mm2_body(degt_ref, b_ref, w_ref, *rest):
    ss = rest[0:NCHUNK]
    mm = rest[NCHUNK:2 * NCHUNK]
    outs = rest[2 * NCHUNK:]
    dinv = _dinv_of(degt_ref[...])
    b = b_ref[...]
    acc = jnp.zeros((_BM, H), jnp.float32)
    for kc in range(NCHUNK):
        h = jnp.maximum(
            (ss[kc][...] + mm[kc][...]) * dinv + b[:, kc * CW:(kc + 1) * CW],
            0.0)
        acc = acc + jnp.dot(h, w_ref[pl.ds(kc * CW, CW), :],
                            preferred_element_type=jnp.float32)
    y = acc * dinv
    for kc, o in enumerate(outs):
        o[...] = y[:, kc * CW:(kc + 1) * CW]


def _mm2(degt, b1r, w2, aggs, ms):
    return pl.pallas_call(
        _mm2_body,
        grid=(N // _BM,),
        in_specs=[
            pl.BlockSpec((_BM, 32), lambda i: (i, 0)),
            pl.BlockSpec((1, H), lambda i: (0, 0)),
            pl.BlockSpec((H, H), lambda i: (0, 0)),
        ] + [pl.BlockSpec((_BM, CW), lambda i: (i, 0))] * (2 * NCHUNK),
        out_specs=[pl.BlockSpec((_BM, CW), lambda i: (i, 0))] * NCHUNK,
        out_shape=[jax.ShapeDtypeStruct((N, CW), jnp.float32)] * NCHUNK,
    )(degt, b1r, w2, *aggs, *ms)


def _mm3_body(degt_ref, b_ref, be1_ref, wa_ref, wb_ref, *rest):
    ss = rest[0:NCHUNK]
    mm = rest[NCHUNK:2 * NCHUNK]
    hs_ref, hd_ref = rest[2 * NCHUNK:]
    dinv = _dinv_of(degt_ref[...])
    b = b_ref[...]
    acca = jnp.zeros((_BM, H), jnp.float32)
    accb = jnp.zeros((_BM, H), jnp.float32)
    for kc in range(NCHUNK):
        h = jnp.maximum(
            (ss[kc][...] + mm[kc][...]) * dinv + b[:, kc * CW:(kc + 1) * CW],
            0.0)
        acca = acca + jnp.dot(h, wa_ref[pl.ds(kc * CW, CW), :],
                              preferred_element_type=jnp.float32)
        accb = accb + jnp.dot(h, wb_ref[pl.ds(kc * CW, CW), :],
                              preferred_element_type=jnp.float32)
    hs_ref[...] = acca
    hd_ref[...] = accb + be1_ref[...]


def _mm3(degt, b2r, be1r, we1a, we1b, aggs, ms):
    return pl.pallas_call(
        _mm3_body,
        grid=(N // _BM,),
        in_specs=[
            pl.BlockSpec((_BM, 32), lambda i: (i, 0)),
            pl.BlockSpec((1, H), lambda i: (0, 0)),
            pl.BlockSpec((1, H), lambda i: (0, 0)),
            pl.BlockSpec((H, H), lambda i: (0, 0)),
            pl.BlockSpec((H, H), lambda i: (0, 0)),
        ] + [pl.BlockSpec((_BM, CW), lambda i: (i, 0))] * (2 * NCHUNK),
        out_specs=[pl.BlockSpec((_BM, H), lambda i: (i, 0))] * 2,
        out_shape=[jax.ShapeDtypeStruct((N, H), jnp.float32)] * 2,
    )(degt, b2r, be1r, we1a, we1b, *aggs, *ms)


@jax.jit
def kernel(x, edge_index, W1, b1, W2, b2, We1, be1, We2, be2):
    E = edge_index.shape[1]
    src = edge_index[0]
    dst = edge_index[1]

    pad = EPAD - E
    srcp = jnp.concatenate([src, jnp.zeros((pad,), jnp.int32)])
    dst_sac = jnp.concatenate([dst, jnp.full((pad,), N, jnp.int32)])
    dst_z = jnp.concatenate([dst, jnp.zeros((pad,), jnp.int32)])
    src16 = srcp.reshape(16, EPS // AB, AB)
    dst16 = dst_sac.reshape(16, EPS // AB, AB)
    src32 = srcp.reshape(32, EPT // 64, 64)
    dst32 = dst_z.reshape(32, EPT // 64, 64)

    hist = _deg_kernel(dst_sac)
    degt = jnp.transpose(hist)

    b1r = b1.reshape(1, H)
    b2r = b2.reshape(1, H)
    be1r = be1.reshape(1, H)
    we1a = We1[:H, :]
    we1b = We1[H:, :]

    m1 = _mm1(x, W1, degt)
    agg1 = _agg_kernel(*m1, src16, dst16)
    m2 = _mm2(degt, b1r, W2, agg1, m1)
    agg2 = _agg_kernel(*m2, src16, dst16)
    hs, hd = _mm3(degt, b2r, be1r, we1a, we1b, agg2, m2)

    we2v = We2.reshape(H)
    be2v = jnp.broadcast_to(be2.reshape(1), (16,))
    scores = _edge_kernel(hs, hd, src32, dst32, we2v, be2v)
    return scores[:E]


# speedup vs baseline: 5.3457x; 1.1589x over previous
import functools

import jax
import jax.numpy as jnp
from jax import lax
from jax.experimental import pallas as pl
from jax.experimental.pallas import tpu as pltpu
from jax.experimental.pallas import tpu_sc as plsc

N = 10000
NP = 10016
D = 256
H = 512
NCHUNK = 4
CW = H // NCHUNK
EPAD = 163840
EPT = EPAD // 32
EPS = EPAD // 16
AB = 128

_mesh = plsc.VectorSubcoreMesh(core_axis_name="c", subcore_axis_name="s")

import dataclasses as _dataclasses
_sc_params = pltpu.CompilerParams()
if "needs_layout_passes" in pltpu.CompilerParams.__dataclass_fields__:
    _sc_params = _dataclasses.replace(_sc_params, needs_layout_passes=False)


@functools.partial(
    pl.kernel,
    out_type=jax.ShapeDtypeStruct((32, NP), jnp.float32),
    mesh=_mesh,
    scratch_types=[
        pltpu.VMEM((EPT,), jnp.int32),
        pltpu.VMEM((NP,), jnp.float32),
    ],
    compiler_params=_sc_params,
)
def _deg_kernel(dst_hbm, out_hbm, idx_v, hist_v):
    c = lax.axis_index("c")
    s = lax.axis_index("s")
    w = s * 2 + c

    @pl.loop(0, NP, step=16)
    def _(i):
        hist_v[pl.ds(i, 16)] = jnp.zeros((16,), jnp.float32)

    pltpu.sync_copy(dst_hbm.at[pl.ds(w * EPT, EPT)], idx_v)
    ones = jnp.ones((16,), jnp.float32)

    @pl.loop(0, EPT, step=16)
    def _(i):
        idx = idx_v[pl.ds(i, 16)]
        plsc.addupdate_scatter(hist_v, [idx], ones)

    pltpu.sync_copy(hist_v, out_hbm.at[w])


def _agg_chunk(m_ref, out_ref, s, acc, src_hbm, dst_hbm, srcv, dstv,
               rowsA, rowsB, sGA, sGB, sSA, sSB):
    @pl.when(s < 15)
    def _():
        off = s * 632

        @pl.loop(0, 4)
        def _(j):
            pltpu.sync_copy(rowsA, acc.at[pl.ds(off + j * 128, 128)])

        pltpu.sync_copy(rowsA.at[pl.ds(0, 120)], acc.at[pl.ds(off + 512, 120)])

    @pl.when(s == 15)
    def _():
        @pl.loop(0, 4)
        def _(j):
            pltpu.sync_copy(rowsA, acc.at[pl.ds(9480 + j * 128, 128)])

        pltpu.sync_copy(rowsA.at[pl.ds(0, 8)], acc.at[pl.ds(9992, 8)])

    plsc.subcore_barrier()

    NBH = EPS // AB // 2
    for half in range(2):
        pltpu.sync_copy(src_hbm.at[s, pl.ds(half * NBH, NBH)], srcv)
        pltpu.sync_copy(dst_hbm.at[s, pl.ds(half * NBH, NBH)], dstv)
        pltpu.async_copy(m_ref.at[srcv.at[0]], rowsA, sGA)

        @pl.loop(0, NBH, step=2)
        def _(b):
            pltpu.make_async_copy(m_ref.at[srcv.at[b]], rowsA, sGA).wait()

            @pl.when(b > 0)
            def _():
                pltpu.make_async_copy(rowsB, acc.at[dstv.at[b]], sSB).wait()

            pltpu.async_copy(m_ref.at[srcv.at[b + 1]], rowsB, sGB)
            pltpu.async_copy(rowsA, acc.at[dstv.at[b]], sSA, add=True)

            pltpu.make_async_copy(m_ref.at[srcv.at[b + 1]], rowsB, sGB).wait()

            @pl.when(b + 2 < NBH)
            def _():
                pltpu.make_async_copy(rowsA, acc.at[dstv.at[b]], sSA).wait()
                pltpu.async_copy(m_ref.at[srcv.at[b + 2]], rowsA, sGA)

            pltpu.async_copy(rowsB, acc.at[dstv.at[b + 1]], sSB, add=True)

        pltpu.make_async_copy(rowsA, acc.at[dstv.at[0]], sSA).wait()
        pltpu.make_async_copy(rowsB, acc.at[dstv.at[0]], sSB).wait()

    plsc.subcore_barrier()

    @pl.when(s < 15)
    def _():
        off = s * 632
        pltpu.sync_copy(acc.at[pl.ds(off, 632)], out_ref.at[pl.ds(off, 632)])

    @pl.when(s == 15)
    def _():
        pltpu.sync_copy(acc.at[pl.ds(9480, 520)], out_ref.at[pl.ds(9480, 520)])

    plsc.subcore_barrier()


def _zero_rows(rowsA):
    @pl.loop(0, AB)
    def _(r):
        @pl.loop(0, CW, step=16)
        def _(j):
            rowsA[r, pl.ds(j, 16)] = jnp.zeros((16,), jnp.float32)


def _make_agg(nch):
    per_core = nch // 2

    @functools.partial(
        pl.kernel,
        out_type=[jax.ShapeDtypeStruct((N, CW), jnp.float32)] * nch,
        mesh=_mesh,
        scratch_types=[
            pltpu.VMEM_SHARED((NP, CW), jnp.float32),
            pltpu.VMEM((EPS // AB // 2, AB), jnp.int32),
            pltpu.VMEM((EPS // AB // 2, AB), jnp.int32),
            pltpu.VMEM((AB, CW), jnp.float32),
            pltpu.VMEM((AB, CW), jnp.float32),
            pltpu.SemaphoreType.DMA,
            pltpu.SemaphoreType.DMA,
            pltpu.SemaphoreType.DMA,
            pltpu.SemaphoreType.DMA,
        ],
        compiler_params=_sc_params,
    )
    def agg(*refs):
        ms = refs[0:nch]
        src_hbm, dst_hbm = refs[nch], refs[nch + 1]
        outs = refs[nch + 2:2 * nch + 2]
        acc, srcv, dstv, rowsA, rowsB, sGA, sGB, sSA, sSB = refs[2 * nch + 2:]
        c = lax.axis_index("c")
        s = lax.axis_index("s")
        _zero_rows(rowsA)

        @pl.when(c == 0)
        def _():
            for k in range(per_core):
                _agg_chunk(ms[k], outs[k], s, acc, src_hbm, dst_hbm, srcv,
                           dstv, rowsA, rowsB, sGA, sGB, sSA, sSB)

        @pl.when(c == 1)
        def _():
            for k in range(per_core, nch):
                _agg_chunk(ms[k], outs[k], s, acc, src_hbm, dst_hbm, srcv,
                           dstv, rowsA, rowsB, sGA, sGB, sSA, sSB)

    return agg


_agg2 = _make_agg(2)
_agg4 = _make_agg(4)


@functools.partial(
    pl.kernel,
    out_type=jax.ShapeDtypeStruct((EPAD,), jnp.float32),
    mesh=_mesh,
    scratch_types=[
        pltpu.VMEM((EPT // 64, 64), jnp.int32),
        pltpu.VMEM((EPT // 64, 64), jnp.int32),
        pltpu.VMEM((64, H), jnp.float32),
        pltpu.VMEM((64, H), jnp.float32),
        pltpu.VMEM((H,), jnp.float32),
        pltpu.VMEM((16,), jnp.float32),
        pltpu.VMEM((EPT,), jnp.float32),
        pltpu.SemaphoreType.DMA,
        pltpu.SemaphoreType.DMA,
        pltpu.SemaphoreType.DMA,
        pltpu.SemaphoreType.DMA,
    ],
    compiler_params=_sc_params,
)
def _edge_kernel(hs_hbm, hd_hbm, src_hbm, dst_hbm, we2_hbm, be2_hbm, out_hbm,
                 srcv, dstv, bufA, bufB, wv, b2v, score_v,
                 sGA, sGB, sHA, sHB):
    c = lax.axis_index("c")
    s = lax.axis_index("s")
    w = s * 2 + c
    pltpu.sync_copy(src_hbm.at[w], srcv)
    pltpu.sync_copy(dst_hbm.at[w], dstv)
    pltpu.sync_copy(we2_hbm, wv)
    pltpu.sync_copy(be2_hbm, b2v)

    wregs = [wv[pl.ds(kc * 16, 16)] for kc in range(H // 16)]
    lane0 = lax.iota(jnp.int32, 16) == 0

    def compute(buf, batch):
        base = batch * 64

        @pl.loop(0, 64)
        def _(e):
            acc = jnp.zeros((16,), jnp.float32)
            for kc in range(H // 16):
                u = jnp.maximum(buf[e, pl.ds(kc * 16, 16)], 0.0)
                acc = acc + u * wregs[kc]
            t = jnp.broadcast_to(jnp.sum(acc), (16,))
            idx = jnp.broadcast_to(base + e, (16,)).astype(jnp.int32)
            plsc.store_scatter(score_v, [idx], t, mask=lane0)

    NB = EPT // 64
    pltpu.async_copy(hs_hbm.at[srcv.at[0]], bufA, sGA)

    @pl.loop(0, NB, step=2)
    def _(b):
        pltpu.make_async_copy(hs_hbm.at[srcv.at[b]], bufA, sGA).wait()
        pltpu.async_copy(hd_hbm.at[dstv.at[b]], bufA, sHA, add=True)
        pltpu.async_copy(hs_hbm.at[srcv.at[b + 1]], bufB, sGB)
        pltpu.make_async_copy(hd_hbm.at[dstv.at[b]], bufA, sHA).wait()
        compute(bufA, b)

        pltpu.make_async_copy(hs_hbm.at[srcv.at[b + 1]], bufB, sGB).wait()
        pltpu.async_copy(hd_hbm.at[dstv.at[b + 1]], bufB, sHB, add=True)

        @pl.when(b + 2 < NB)
        def _():
            pltpu.async_copy(hs_hbm.at[srcv.at[b + 2]], bufA, sGA)

        pltpu.make_async_copy(hd_hbm.at[dstv.at[b + 1]], bufB, sHB).wait()
        compute(bufB, b + 1)

    b2 = b2v[pl.ds(0, 16)]

    @pl.loop(0, EPT, step=16)
    def _(i):
        t = score_v[pl.ds(i, 16)] + b2
        score_v[pl.ds(i, 16)] = 1.0 / (1.0 + jnp.exp(-t))

    pltpu.sync_copy(score_v, out_hbm.at[pl.ds(w * EPT, EPT)])


_BM = 1000


def _dinv_of(degt_blk):
    deg = jnp.sum(degt_blk, axis=1, keepdims=True) + 1.0
    return lax.rsqrt(deg)


def _mm0_body(x_ref, degt_ref, *outs):
    dinv = _dinv_of(degt_ref[...])
    y = x_ref[...] * dinv
    for kc, o in enumerate(outs):
        o[...] = y[:, kc * CW:(kc + 1) * CW]


def _mm0(x, degt):
    nd = D // CW
    return pl.pallas_call(
        _mm0_body,
        grid=(N // _BM,),
        in_specs=[
            pl.BlockSpec((_BM, D), lambda i: (i, 0)),
            pl.BlockSpec((_BM, 32), lambda i: (i, 0)),
        ],
        out_specs=[pl.BlockSpec((_BM, CW), lambda i: (i, 0))] * nd,
        out_shape=[jax.ShapeDtypeStruct((N, CW), jnp.float32)] * nd,
    )(x, degt)


def _mm12_body(degt_ref, b_ref, w1_ref, w2_ref, *rest):
    nd = D // CW
    aa = rest[0:nd]
    xx = rest[nd:2 * nd]
    outs = rest[2 * nd:]
    dinv = _dinv_of(degt_ref[...])
    y1 = jnp.zeros((_BM, H), jnp.float32)
    for kc in range(nd):
        y1 = y1 + jnp.dot(aa[kc][...] + xx[kc][...],
                          w1_ref[pl.ds(kc * CW, CW), :],
                          preferred_element_type=jnp.float32)
    h1 = jnp.maximum(y1 * dinv + b_ref[...], 0.0)
    y2 = jnp.dot(h1, w2_ref[...], preferred_element_type=jnp.float32)
    y = y2 * dinv
    for kc, o in enumerate(outs):
        o[...] = y[:, kc * CW:(kc + 1) * CW]


def _mm12(degt, b1r, w1, w2, aggs, xds):
    nd = D // CW
    return pl.pallas_call(
        _mm12_body,
        grid=(N // _BM,),
        in_specs=[
            pl.BlockSpec((_BM, 32), lambda i: (i, 0)),
            pl.BlockSpec((1, H), lambda i: (0, 0)),
            pl.BlockSpec((D, H), lambda i: (0, 0)),
            pl.BlockSpec((H, H), lambda i: (0, 0)),
        ] + [pl.BlockSpec((_BM, CW), lambda i: (i, 0))] * (2 * nd),
        out_specs=[pl.BlockSpec((_BM, CW), lambda i: (i, 0))] * NCHUNK,
        out_shape=[jax.ShapeDtypeStruct((N, CW), jnp.float32)] * NCHUNK,
    )(degt, b1r, w1, w2, *aggs, *xds)


def _mm3_body(degt_ref, b_ref, be1_ref, wa_ref, wb_ref, *rest):
    ss = rest[0:NCHUNK]
    mm = rest[NCHUNK:2 * NCHUNK]
    hs_ref, hd_ref = rest[2 * NCHUNK:]
    dinv = _dinv_of(degt_ref[...])
    b = b_ref[...]
    acca = jnp.zeros((_BM, H), jnp.float32)
    accb = jnp.zeros((_BM, H), jnp.float32)
    for kc in range(NCHUNK):
        h = jnp.maximum(
            (ss[kc][...] + mm[kc][...]) * dinv + b[:, kc * CW:(kc + 1) * CW],
            0.0)
        acca = acca + jnp.dot(h, wa_ref[pl.ds(kc * CW, CW), :],
                              preferred_element_type=jnp.float32)
        accb = accb + jnp.dot(h, wb_ref[pl.ds(kc * CW, CW), :],
                              preferred_element_type=jnp.float32)
    hs_ref[...] = acca
    hd_ref[...] = accb + be1_ref[...]


def _mm3(degt, b2r, be1r, we1a, we1b, aggs, ms):
    return pl.pallas_call(
        _mm3_body,
        grid=(N // _BM,),
        in_specs=[
            pl.BlockSpec((_BM, 32), lambda i: (i, 0)),
            pl.BlockSpec((1, H), lambda i: (0, 0)),
            pl.BlockSpec((1, H), lambda i: (0, 0)),
            pl.BlockSpec((H, H), lambda i: (0, 0)),
            pl.BlockSpec((H, H), lambda i: (0, 0)),
        ] + [pl.BlockSpec((_BM, CW), lambda i: (i, 0))] * (2 * NCHUNK),
        out_specs=[pl.BlockSpec((_BM, H), lambda i: (i, 0))] * 2,
        out_shape=[jax.ShapeDtypeStruct((N, H), jnp.float32)] * 2,
    )(degt, b2r, be1r, we1a, we1b, *aggs, *ms)


@jax.jit
def kernel(x, edge_index, W1, b1, W2, b2, We1, be1, We2, be2):
    E = edge_index.shape[1]
    src = edge_index[0]
    dst = edge_index[1]

    pad = EPAD - E
    srcp = jnp.concatenate([src, jnp.zeros((pad,), jnp.int32)])
    dst_sac = jnp.concatenate([dst, jnp.full((pad,), N, jnp.int32)])
    dst_z = jnp.concatenate([dst, jnp.zeros((pad,), jnp.int32)])
    src16 = srcp.reshape(16, EPS // AB, AB)
    dst16 = dst_sac.reshape(16, EPS // AB, AB)
    src32 = srcp.reshape(32, EPT // 64, 64)
    dst32 = dst_z.reshape(32, EPT // 64, 64)

    hist = _deg_kernel(dst_sac)
    degt = jnp.transpose(hist)

    b1r = b1.reshape(1, H)
    b2r = b2.reshape(1, H)
    be1r = be1.reshape(1, H)
    we1a = We1[:H, :]
    we1b = We1[H:, :]

    xd = _mm0(x, degt)
    agg1 = _agg2(*xd, src16, dst16)
    m2 = _mm12(degt, b1r, W1, W2, agg1, xd)
    agg2 = _agg4(*m2, src16, dst16)
    hs, hd = _mm3(degt, b2r, be1r, we1a, we1b, agg2, m2)

    we2v = We2.reshape(H)
    be2v = jnp.broadcast_to(be2.reshape(1), (16,))
    scores = _edge_kernel(hs, hd, src32, dst32, we2v, be2v)
    return scores[:E]


# speedup vs baseline: 5.5917x; 1.0460x over previous
import functools

import jax
import jax.numpy as jnp
from jax import lax
from jax.experimental import pallas as pl
from jax.experimental.pallas import tpu as pltpu
from jax.experimental.pallas import tpu_sc as plsc

N = 10000
NP = 10016
D = 256
H = 512
NCHUNK = 4
CW = H // NCHUNK
EPAD = 163840
EPT = EPAD // 32
EPS = EPAD // 16
AB = 128

_mesh = plsc.VectorSubcoreMesh(core_axis_name="c", subcore_axis_name="s")

import dataclasses as _dataclasses
_sc_params = pltpu.CompilerParams()
if "needs_layout_passes" in pltpu.CompilerParams.__dataclass_fields__:
    _sc_params = _dataclasses.replace(_sc_params, needs_layout_passes=False)


@functools.partial(
    pl.kernel,
    out_type=jax.ShapeDtypeStruct((32, NP), jnp.float32),
    mesh=_mesh,
    scratch_types=[
        pltpu.VMEM((EPT,), jnp.int32),
        pltpu.VMEM((NP,), jnp.float32),
    ],
    compiler_params=_sc_params,
)
def _deg_kernel(dst_hbm, out_hbm, idx_v, hist_v):
    c = lax.axis_index("c")
    s = lax.axis_index("s")
    w = s * 2 + c

    @pl.loop(0, NP, step=16)
    def _(i):
        hist_v[pl.ds(i, 16)] = jnp.zeros((16,), jnp.float32)

    pltpu.sync_copy(dst_hbm.at[pl.ds(w * EPT, EPT)], idx_v)
    ones = jnp.ones((16,), jnp.float32)

    @pl.loop(0, EPT, step=16)
    def _(i):
        idx = idx_v[pl.ds(i, 16)]
        plsc.addupdate_scatter(hist_v, [idx], ones)

    pltpu.sync_copy(hist_v, out_hbm.at[w])


def _agg_chunk(m_ref, out_ref, s, acc, src_hbm, dst_hbm, srcv, dstv,
               rowsA, rowsB, sGA, sGB, sSA, sSB):
    @pl.when(s < 15)
    def _():
        off = s * 632

        @pl.loop(0, 4)
        def _(j):
            pltpu.sync_copy(rowsA, acc.at[pl.ds(off + j * 128, 128)])

        pltpu.sync_copy(rowsA.at[pl.ds(0, 120)], acc.at[pl.ds(off + 512, 120)])

    @pl.when(s == 15)
    def _():
        @pl.loop(0, 4)
        def _(j):
            pltpu.sync_copy(rowsA, acc.at[pl.ds(9480 + j * 128, 128)])

        pltpu.sync_copy(rowsA.at[pl.ds(0, 8)], acc.at[pl.ds(9992, 8)])

    plsc.subcore_barrier()

    NBH = EPS // AB // 2
    for half in range(2):
        pltpu.sync_copy(src_hbm.at[s, pl.ds(half * NBH, NBH)], srcv)
        pltpu.sync_copy(dst_hbm.at[s, pl.ds(half * NBH, NBH)], dstv)
        pltpu.async_copy(m_ref.at[srcv.at[0]], rowsA, sGA)

        @pl.loop(0, NBH, step=2)
        def _(b):
            pltpu.make_async_copy(m_ref.at[srcv.at[b]], rowsA, sGA).wait()

            @pl.when(b > 0)
            def _():
                pltpu.make_async_copy(rowsB, acc.at[dstv.at[b]], sSB).wait()

            pltpu.async_copy(m_ref.at[srcv.at[b + 1]], rowsB, sGB)
            pltpu.async_copy(rowsA, acc.at[dstv.at[b]], sSA, add=True)

            pltpu.make_async_copy(m_ref.at[srcv.at[b + 1]], rowsB, sGB).wait()

            @pl.when(b + 2 < NBH)
            def _():
                pltpu.make_async_copy(rowsA, acc.at[dstv.at[b]], sSA).wait()
                pltpu.async_copy(m_ref.at[srcv.at[b + 2]], rowsA, sGA)

            pltpu.async_copy(rowsB, acc.at[dstv.at[b + 1]], sSB, add=True)

        pltpu.make_async_copy(rowsA, acc.at[dstv.at[0]], sSA).wait()
        pltpu.make_async_copy(rowsB, acc.at[dstv.at[0]], sSB).wait()

    plsc.subcore_barrier()

    @pl.when(s < 15)
    def _():
        off = s * 632
        pltpu.sync_copy(acc.at[pl.ds(off, 632)], out_ref.at[pl.ds(off, 632)])

    @pl.when(s == 15)
    def _():
        pltpu.sync_copy(acc.at[pl.ds(9480, 520)], out_ref.at[pl.ds(9480, 520)])

    plsc.subcore_barrier()


def _zero_rows(rowsA):
    @pl.loop(0, AB)
    def _(r):
        @pl.loop(0, CW, step=16)
        def _(j):
            rowsA[r, pl.ds(j, 16)] = jnp.zeros((16,), jnp.float32)


def _make_agg(nch):
    per_core = nch // 2

    @functools.partial(
        pl.kernel,
        out_type=[jax.ShapeDtypeStruct((N, CW), jnp.float32)] * nch,
        mesh=_mesh,
        scratch_types=[
            pltpu.VMEM_SHARED((NP, CW), jnp.float32),
            pltpu.VMEM((EPS // AB // 2, AB), jnp.int32),
            pltpu.VMEM((EPS // AB // 2, AB), jnp.int32),
            pltpu.VMEM((AB, CW), jnp.float32),
            pltpu.VMEM((AB, CW), jnp.float32),
            pltpu.SemaphoreType.DMA,
            pltpu.SemaphoreType.DMA,
            pltpu.SemaphoreType.DMA,
            pltpu.SemaphoreType.DMA,
        ],
        compiler_params=_sc_params,
    )
    def agg(*refs):
        ms = refs[0:nch]
        src_hbm, dst_hbm = refs[nch], refs[nch + 1]
        outs = refs[nch + 2:2 * nch + 2]
        acc, srcv, dstv, rowsA, rowsB, sGA, sGB, sSA, sSB = refs[2 * nch + 2:]
        c = lax.axis_index("c")
        s = lax.axis_index("s")
        _zero_rows(rowsA)

        @pl.when(c == 0)
        def _():
            for k in range(per_core):
                _agg_chunk(ms[k], outs[k], s, acc, src_hbm, dst_hbm, srcv,
                           dstv, rowsA, rowsB, sGA, sGB, sSA, sSB)

        @pl.when(c == 1)
        def _():
            for k in range(per_core, nch):
                _agg_chunk(ms[k], outs[k], s, acc, src_hbm, dst_hbm, srcv,
                           dstv, rowsA, rowsB, sGA, sGB, sSA, sSB)

    return agg


_agg2 = _make_agg(2)
_agg4 = _make_agg(4)


@functools.partial(
    pl.kernel,
    out_type=jax.ShapeDtypeStruct((EPAD,), jnp.float32),
    mesh=_mesh,
    scratch_types=[
        pltpu.VMEM((EPT // 64, 64), jnp.int32),
        pltpu.VMEM((EPT // 64, 64), jnp.int32),
        pltpu.VMEM((64, H), jnp.float32),
        pltpu.VMEM((64, H), jnp.float32),
        pltpu.VMEM((H,), jnp.float32),
        pltpu.VMEM((16,), jnp.float32),
        pltpu.VMEM((EPT,), jnp.float32),
        pltpu.SemaphoreType.DMA,
        pltpu.SemaphoreType.DMA,
        pltpu.SemaphoreType.DMA,
        pltpu.SemaphoreType.DMA,
    ],
    compiler_params=_sc_params,
)
def _edge_kernel(hs_hbm, hd_hbm, src_hbm, dst_hbm, we2_hbm, be2_hbm, out_hbm,
                 srcv, dstv, bufA, bufB, wv, b2v, score_v,
                 sGA, sGB, sHA, sHB):
    c = lax.axis_index("c")
    s = lax.axis_index("s")
    w = s * 2 + c
    pltpu.sync_copy(src_hbm.at[w], srcv)
    pltpu.sync_copy(dst_hbm.at[w], dstv)
    pltpu.sync_copy(we2_hbm, wv)
    pltpu.sync_copy(be2_hbm, b2v)

    wregs = [wv[pl.ds(kc * 16, 16)] for kc in range(H // 16)]
    lane0 = lax.iota(jnp.int32, 16) == 0

    def compute(buf, batch):
        base = batch * 64

        def edge_dot(e):
            accs = [jnp.zeros((16,), jnp.float32) for _ in range(4)]
            for kc in range(H // 16):
                u = jnp.maximum(buf[e, pl.ds(kc * 16, 16)], 0.0)
                accs[kc % 4] = accs[kc % 4] + u * wregs[kc]
            return jnp.sum((accs[0] + accs[1]) + (accs[2] + accs[3]))

        @pl.loop(0, 64, step=2)
        def _(e):
            t0 = edge_dot(e)
            t1 = edge_dot(e + 1)
            idx0 = jnp.broadcast_to(base + e, (16,)).astype(jnp.int32)
            plsc.store_scatter(score_v, [idx0],
                               jnp.broadcast_to(t0, (16,)), mask=lane0)
            idx1 = jnp.broadcast_to(base + e + 1, (16,)).astype(jnp.int32)
            plsc.store_scatter(score_v, [idx1],
                               jnp.broadcast_to(t1, (16,)), mask=lane0)

    NB = EPT // 64
    pltpu.async_copy(hs_hbm.at[srcv.at[0]], bufA, sGA)

    @pl.loop(0, NB, step=2)
    def _(b):
        pltpu.make_async_copy(hs_hbm.at[srcv.at[b]], bufA, sGA).wait()
        pltpu.async_copy(hd_hbm.at[dstv.at[b]], bufA, sHA, add=True)
        pltpu.async_copy(hs_hbm.at[srcv.at[b + 1]], bufB, sGB)
        pltpu.make_async_copy(hd_hbm.at[dstv.at[b]], bufA, sHA).wait()
        compute(bufA, b)

        pltpu.make_async_copy(hs_hbm.at[srcv.at[b + 1]], bufB, sGB).wait()
        pltpu.async_copy(hd_hbm.at[dstv.at[b + 1]], bufB, sHB, add=True)

        @pl.when(b + 2 < NB)
        def _():
            pltpu.async_copy(hs_hbm.at[srcv.at[b + 2]], bufA, sGA)

        pltpu.make_async_copy(hd_hbm.at[dstv.at[b + 1]], bufB, sHB).wait()
        compute(bufB, b + 1)

    b2 = b2v[pl.ds(0, 16)]

    @pl.loop(0, EPT, step=16)
    def _(i):
        t = score_v[pl.ds(i, 16)] + b2
        score_v[pl.ds(i, 16)] = 1.0 / (1.0 + jnp.exp(-t))

    pltpu.sync_copy(score_v, out_hbm.at[pl.ds(w * EPT, EPT)])


_BM = 1000


def _dinv_of(degt_blk):
    deg = jnp.sum(degt_blk, axis=1, keepdims=True) + 1.0
    return lax.rsqrt(deg)


def _mm0_body(x_ref, degt_ref, *outs):
    dinv = _dinv_of(degt_ref[...])
    y = x_ref[...] * dinv
    for kc, o in enumerate(outs):
        o[...] = y[:, kc * CW:(kc + 1) * CW]


def _mm0(x, degt):
    nd = D // CW
    return pl.pallas_call(
        _mm0_body,
        grid=(N // _BM,),
        in_specs=[
            pl.BlockSpec((_BM, D), lambda i: (i, 0)),
            pl.BlockSpec((_BM, 32), lambda i: (i, 0)),
        ],
        out_specs=[pl.BlockSpec((_BM, CW), lambda i: (i, 0))] * nd,
        out_shape=[jax.ShapeDtypeStruct((N, CW), jnp.float32)] * nd,
    )(x, degt)


def _mm12_body(degt_ref, b_ref, w1_ref, w2_ref, *rest):
    nd = D // CW
    aa = rest[0:nd]
    xx = rest[nd:2 * nd]
    outs = rest[2 * nd:]
    dinv = _dinv_of(degt_ref[...])
    y1 = jnp.zeros((_BM, H), jnp.float32)
    for kc in range(nd):
        y1 = y1 + jnp.dot(aa[kc][...] + xx[kc][...],
                          w1_ref[pl.ds(kc * CW, CW), :],
                          preferred_element_type=jnp.float32)
    h1 = jnp.maximum(y1 * dinv + b_ref[...], 0.0)
    y2 = jnp.dot(h1, w2_ref[...], preferred_element_type=jnp.float32)
    y = y2 * dinv
    for kc, o in enumerate(outs):
        o[...] = y[:, kc * CW:(kc + 1) * CW]


def _mm12(degt, b1r, w1, w2, aggs, xds):
    nd = D // CW
    return pl.pallas_call(
        _mm12_body,
        grid=(N // _BM,),
        in_specs=[
            pl.BlockSpec((_BM, 32), lambda i: (i, 0)),
            pl.BlockSpec((1, H), lambda i: (0, 0)),
            pl.BlockSpec((D, H), lambda i: (0, 0)),
            pl.BlockSpec((H, H), lambda i: (0, 0)),
        ] + [pl.BlockSpec((_BM, CW), lambda i: (i, 0))] * (2 * nd),
        out_specs=[pl.BlockSpec((_BM, CW), lambda i: (i, 0))] * NCHUNK,
        out_shape=[jax.ShapeDtypeStruct((N, CW), jnp.float32)] * NCHUNK,
    )(degt, b1r, w1, w2, *aggs, *xds)


def _mm3_body(degt_ref, b_ref, be1_ref, wa_ref, wb_ref, *rest):
    ss = rest[0:NCHUNK]
    mm = rest[NCHUNK:2 * NCHUNK]
    hs_ref, hd_ref = rest[2 * NCHUNK:]
    dinv = _dinv_of(degt_ref[...])
    b = b_ref[...]
    acca = jnp.zeros((_BM, H), jnp.float32)
    accb = jnp.zeros((_BM, H), jnp.float32)
    for kc in range(NCHUNK):
        h = jnp.maximum(
            (ss[kc][...] + mm[kc][...]) * dinv + b[:, kc * CW:(kc + 1) * CW],
            0.0)
        acca = acca + jnp.dot(h, wa_ref[pl.ds(kc * CW, CW), :],
                              preferred_element_type=jnp.float32)
        accb = accb + jnp.dot(h, wb_ref[pl.ds(kc * CW, CW), :],
                              preferred_element_type=jnp.float32)
    hs_ref[...] = acca
    hd_ref[...] = accb + be1_ref[...]


def _mm3(degt, b2r, be1r, we1a, we1b, aggs, ms):
    return pl.pallas_call(
        _mm3_body,
        grid=(N // _BM,),
        in_specs=[
            pl.BlockSpec((_BM, 32), lambda i: (i, 0)),
            pl.BlockSpec((1, H), lambda i: (0, 0)),
            pl.BlockSpec((1, H), lambda i: (0, 0)),
            pl.BlockSpec((H, H), lambda i: (0, 0)),
            pl.BlockSpec((H, H), lambda i: (0, 0)),
        ] + [pl.BlockSpec((_BM, CW), lambda i: (i, 0))] * (2 * NCHUNK),
        out_specs=[pl.BlockSpec((_BM, H), lambda i: (i, 0))] * 2,
        out_shape=[jax.ShapeDtypeStruct((N, H), jnp.float32)] * 2,
    )(degt, b2r, be1r, we1a, we1b, *aggs, *ms)


@jax.jit
def kernel(x, edge_index, W1, b1, W2, b2, We1, be1, We2, be2):
    E = edge_index.shape[1]
    src = edge_index[0]
    dst = edge_index[1]

    pad = EPAD - E
    srcp = jnp.concatenate([src, jnp.zeros((pad,), jnp.int32)])
    dst_sac = jnp.concatenate([dst, jnp.full((pad,), N, jnp.int32)])
    dst_z = jnp.concatenate([dst, jnp.zeros((pad,), jnp.int32)])
    src16 = srcp.reshape(16, EPS // AB, AB)
    dst16 = dst_sac.reshape(16, EPS // AB, AB)
    src32 = srcp.reshape(32, EPT // 64, 64)
    dst32 = dst_z.reshape(32, EPT // 64, 64)

    hist = _deg_kernel(dst_sac)
    degt = jnp.transpose(hist)

    b1r = b1.reshape(1, H)
    b2r = b2.reshape(1, H)
    be1r = be1.reshape(1, H)
    we1a = We1[:H, :]
    we1b = We1[H:, :]

    xd = _mm0(x, degt)
    agg1 = _agg2(*xd, src16, dst16)
    m2 = _mm12(degt, b1r, W1, W2, agg1, xd)
    agg2 = _agg4(*m2, src16, dst16)
    hs, hd = _mm3(degt, b2r, be1r, we1a, we1b, agg2, m2)

    we2v = We2.reshape(H)
    be2v = jnp.broadcast_to(be2.reshape(1), (16,))
    scores = _edge_kernel(hs, hd, src32, dst32, we2v, be2v)
    return scores[:E]
